# Initial kernel scaffold; baseline (speedup 1.0000x reference)
import jax
import jax.numpy as jnp
from jax.experimental import pallas as pl


def kernel(h_dynamic, h_static, W_skvl, W_dkvl, W_q, current_nodes, neigh_idx, neigh_len):
    raise NotImplementedError("write your pallas kernel here")


# speedup vs baseline: 6.1330x; 6.1330x over previous
import functools
import math

import jax
import jax.numpy as jnp
from jax import lax
from jax.experimental import pallas as pl
from jax.experimental.pallas import tpu as pltpu
from jax.experimental.pallas import tpu_sc as plsc


def _kproj_body(hs_ref, hd_ref, wks_ref, wkd_ref, out_ref):
    out_ref[...] = (
        jnp.dot(hs_ref[...].astype(jnp.bfloat16),
                wks_ref[...].astype(jnp.bfloat16),
                preferred_element_type=jnp.float32)
        + jnp.dot(hd_ref[...].astype(jnp.bfloat16),
                  wkd_ref[...].astype(jnp.bfloat16),
                  preferred_element_type=jnp.float32)
    )


def _kproj(h_s, h_d, wks, wkd, block_n=512):
    n, h = h_s.shape
    grid = (n // block_n,)
    return pl.pallas_call(
        _kproj_body,
        grid=grid,
        in_specs=[
            pl.BlockSpec((block_n, h), lambda i: (i, 0)),
            pl.BlockSpec((block_n, h), lambda i: (i, 0)),
            pl.BlockSpec((h, h), lambda i: (0, 0)),
            pl.BlockSpec((h, h), lambda i: (0, 0)),
        ],
        out_specs=pl.BlockSpec((block_n, h), lambda i: (i, 0)),
        out_shape=jax.ShapeDtypeStruct((n, h), jnp.float32),
    )(h_s, h_d, wks, wkd)


def _sc_gather(table, idx_flat, chunk=128):
    n_rows = idx_flat.shape[0]
    h = table.shape[1]
    info = plsc.get_sparse_core_info()
    nw = info.num_cores * info.num_subcores
    rows_per_w = n_rows // nw
    n_chunks = rows_per_w // chunk
    mesh = plsc.VectorSubcoreMesh(core_axis_name="c", subcore_axis_name="s")

    @functools.partial(
        pl.kernel,
        mesh=mesh,
        out_type=jax.ShapeDtypeStruct((n_rows, h), jnp.float32),
        scratch_types=[
            pltpu.VMEM((chunk,), jnp.int32),
            pltpu.VMEM((chunk, h), jnp.float32),
            pltpu.SemaphoreType.DMA,
        ],
    )
    def gather_kernel(table_hbm, idx_hbm, out_hbm, idx_v, rows_v, sem):
        wid = lax.axis_index("s") * info.num_cores + lax.axis_index("c")
        base = wid * rows_per_w

        def body(i, carry):
            off = base + i * chunk
            pltpu.sync_copy(idx_hbm.at[pl.ds(off, chunk)], idx_v)
            pltpu.async_copy(table_hbm.at[idx_v], rows_v, sem).wait()
            pltpu.sync_copy(rows_v, out_hbm.at[pl.ds(off, chunk)])
            return carry

        lax.fori_loop(0, n_chunks, body, 0)

    return gather_kernel(table, idx_flat)


def _attn_body(ki_ref, hq_ref, wq_ref, g_ref, nidx_ref, nlen_ref, cur_ref,
               act_ref, lp_ref, *, m, h):
    bb = hq_ref.shape[0]
    q = jnp.dot(hq_ref[...].astype(jnp.bfloat16),
                wq_ref[...].astype(jnp.bfloat16),
                preferred_element_type=jnp.float32)
    ki = ki_ref[...].astype(jnp.bfloat16).astype(jnp.float32)
    qr = q.astype(jnp.bfloat16).astype(jnp.float32)
    compat = jnp.sum(ki * qr[:, None, :], axis=-1) / math.sqrt(h)
    nlen = nlen_ref[...]
    lane = lax.broadcasted_iota(jnp.int32, (bb, m), 1)
    mask = lane < nlen
    logits = jnp.where(mask, compat, -1e9)
    z = logits + g_ref[...]
    idx = jnp.argmax(z, axis=1)
    mx = jnp.max(logits, axis=1, keepdims=True)
    shifted = logits - mx
    logp_all = shifted - jnp.log(jnp.sum(jnp.exp(shifted), axis=1, keepdims=True))
    sel = lane == idx[:, None]
    logp = jnp.sum(jnp.where(sel, logp_all, 0.0), axis=1)
    chosen = jnp.sum(jnp.where(sel, nidx_ref[...], 0), axis=1)
    empty = nlen[:, 0] == 0
    act_ref[...] = jnp.where(empty, cur_ref[...][:, 0], chosen)[:, None]
    lp_ref[...] = jnp.where(empty, 0.0, logp)[:, None]


def _attn_sample(ki, hq, wq, gumbel, neigh_idx, neigh_len, current_nodes,
                 block_b=256):
    b, m = neigh_idx.shape
    h = hq.shape[1]
    grid = (b // block_b,)
    return pl.pallas_call(
        functools.partial(_attn_body, m=m, h=h),
        grid=grid,
        in_specs=[
            pl.BlockSpec((block_b, m, h), lambda i: (i, 0, 0)),
            pl.BlockSpec((block_b, h), lambda i: (i, 0)),
            pl.BlockSpec((h, h), lambda i: (0, 0)),
            pl.BlockSpec((block_b, m), lambda i: (i, 0)),
            pl.BlockSpec((block_b, m), lambda i: (i, 0)),
            pl.BlockSpec((block_b, 1), lambda i: (i, 0)),
            pl.BlockSpec((block_b, 1), lambda i: (i, 0)),
        ],
        out_specs=[
            pl.BlockSpec((block_b, 1), lambda i: (i, 0)),
            pl.BlockSpec((block_b, 1), lambda i: (i, 0)),
        ],
        out_shape=[
            jax.ShapeDtypeStruct((b, 1), jnp.int32),
            jax.ShapeDtypeStruct((b, 1), jnp.float32),
        ],
    )(ki, hq, wq, gumbel, neigh_idx, neigh_len, current_nodes)


def kernel(h_dynamic, h_static, W_skvl, W_dkvl, W_q, current_nodes, neigh_idx,
           neigh_len):
    n, h = h_static.shape
    b, m = neigh_idx.shape
    wks = W_skvl[:, :h]
    wkd = W_dkvl[:, :h]

    kall = _kproj(h_static, h_dynamic, wks, wkd)

    idx_flat = neigh_idx.reshape(b * m).astype(jnp.int32)
    ki = _sc_gather(kall, idx_flat).reshape(b, m, h)

    cur = current_nodes.astype(jnp.int32)
    hq = h_static[cur] + h_dynamic[cur]

    u = jax.random.uniform(jax.random.key(42), (b, m), minval=1e-9, maxval=1.0)
    gumbel = -jnp.log(-jnp.log(u))

    actions2, logp2 = _attn_sample(
        ki, hq, W_q, gumbel,
        neigh_idx.astype(jnp.int32),
        neigh_len.reshape(b, 1).astype(jnp.int32),
        cur.reshape(b, 1),
    )
    return actions2[:, 0], logp2[:, 0]
